# baseline (speedup 1.0000x reference)
import functools

import jax
import jax.numpy as jnp
from jax import lax
from jax.experimental import pallas as pl
from jax.experimental.pallas import tpu as pltpu
from jax.experimental.pallas import tpu_sc as plsc

_NC = 2
_NS = 16
_NW = _NC * _NS

_VOCAB = 1000000
_D = 64
_SEQ = 200
_BATCH = 4096
_N = _BATCH * _SEQ
_PER_W = _N // _NW
_C = 400
_NCHUNK = _PER_W // _C


def _emb_kernel(ids_hbm, table_hbm, pos_hbm, out_hbm, idx_v, rows_v, pos_v, sem):
    wid = lax.axis_index("s") * _NC + lax.axis_index("c")
    base = wid * _PER_W

    pltpu.sync_copy(pos_hbm, pos_v)

    def chunk_body(c, carry):
        row0 = base + c * _C
        pltpu.sync_copy(ids_hbm.at[pl.ds(row0, _C)], idx_v)
        pltpu.async_copy(table_hbm.at[idx_v], rows_v, sem).wait()

        def row_body(r, carry2):
            pr = lax.select(r < _SEQ, r, r - _SEQ)
            for d in range(_D // 16):
                sl = pl.ds(d * 16, 16)
                rows_v[r, sl] = rows_v[r, sl] + pos_v[pr, sl]
            return carry2

        lax.fori_loop(0, _C, row_body, 0, unroll=2)

        pltpu.sync_copy(rows_v, out_hbm.at[pl.ds(row0, _C)])
        return carry

    lax.fori_loop(0, _NCHUNK, chunk_body, 0)


@jax.jit
def _run(ids_flat, token_table, pos_table):
    mesh = plsc.VectorSubcoreMesh(core_axis_name="c", subcore_axis_name="s")
    return pl.kernel(
        _emb_kernel,
        out_type=jax.ShapeDtypeStruct((_N, _D), jnp.float32),
        mesh=mesh,
        scratch_types=[
            pltpu.VMEM((_C,), jnp.int32),
            pltpu.VMEM((_C, _D), jnp.float32),
            pltpu.VMEM((_SEQ, _D), jnp.float32),
            pltpu.SemaphoreType.DMA,
        ],
        compiler_params=pltpu.CompilerParams(use_tc_tiling_on_sc=False),
    )(ids_flat, token_table, pos_table)


def kernel(token_ids, token_table, pos_table):
    b, s = token_ids.shape
    ids_flat = token_ids.reshape(b * s).astype(jnp.int32)
    out = _run(ids_flat, token_table, pos_table)
    return out.reshape(b, s, _D)
